# Initial kernel scaffold; baseline (speedup 1.0000x reference)
import jax
import jax.numpy as jnp
from jax.experimental import pallas as pl


def kernel(x, edge_index, W, b, gamma, beta):
    raise NotImplementedError("write your pallas kernel here")


# speedup vs baseline: 2.4006x; 2.4006x over previous
import functools

import jax
import jax.numpy as jnp
from jax import lax
from jax.experimental import pallas as pl
from jax.experimental.pallas import tpu as pltpu
from jax.experimental.pallas import tpu_sc as plsc

B = 8
N = 10000
E = 320000
D = 128

NC = 2
NS = 16
G = 128
EPAD = 327680
ROWS = EPAD // G
CPT = ROWS // NS
DEG_CPT = ROWS // NC // NS
NP = 10112
RPT = NP // NS
BPC = B // NC

_mesh = plsc.VectorSubcoreMesh(core_axis_name="c", subcore_axis_name="s",
                               num_cores=NC, num_subcores=NS)


@functools.partial(
    pl.kernel,
    out_type=jax.ShapeDtypeStruct((NC * NS, RPT, 128), jnp.float32),
    mesh=_mesh,
    scratch_types=[
        pltpu.VMEM((8, G), jnp.int32),
        pltpu.VMEM((G, 128), jnp.float32),
        pltpu.VMEM_SHARED((NP, 128), jnp.float32),
    ],
)
def _deg_kernel(src2d_hbm, ones_hbm, zrow_hbm, degp_hbm, srcbuf, onesbuf,
                hist):
    c = lax.axis_index("c")
    s = lax.axis_index("s")
    w = c * NS + s
    pltpu.sync_copy(zrow_hbm, hist.at[pl.ds(s * RPT, RPT)])
    base = c * (NS * DEG_CPT) + s * DEG_CPT
    pltpu.sync_copy(ones_hbm, onesbuf)
    plsc.subcore_barrier()

    def body(grp, carry):
        pltpu.sync_copy(src2d_hbm.at[pl.ds(base + grp * 8, 8)], srcbuf)
        for j in range(8):
            pltpu.sync_copy(onesbuf, hist.at[srcbuf.at[j]], add=True)
        return carry

    lax.fori_loop(0, DEG_CPT // 8, body, 0, unroll=False)
    plsc.subcore_barrier()
    pltpu.sync_copy(hist.at[pl.ds(s * RPT, RPT)], degp_hbm.at[w])


@functools.partial(
    pl.kernel,
    out_type=jax.ShapeDtypeStruct((B * NS, RPT, D), jnp.float32),
    mesh=_mesh,
    scratch_types=[
        pltpu.VMEM((8, G), jnp.int32),
        pltpu.VMEM((8, G), jnp.int32),
        pltpu.VMEM((G,), jnp.int32),
        pltpu.VMEM((G, D), jnp.float32),
        pltpu.VMEM_SHARED((NP, D), jnp.float32),
    ],
)
def _agg_kernel(xs_hbm, src2d_hbm, dst2d_hbm, zeros_hbm, neigh_hbm,
                srcbuf, dstbuf, idxbuf, rows, accum):
    c = lax.axis_index("c")
    s = lax.axis_index("s")

    for k in range(BPC):
        b = c * BPC + k
        boff = b * N
        pltpu.sync_copy(zeros_hbm, accum.at[pl.ds(s * RPT, RPT)])
        plsc.subcore_barrier()

        def body(grp, carry):
            row0 = s * CPT + grp * 8
            pltpu.sync_copy(src2d_hbm.at[pl.ds(row0, 8)], srcbuf)
            pltpu.sync_copy(dst2d_hbm.at[pl.ds(row0, 8)], dstbuf)
            for j in range(8):
                for g in range(G // 16):
                    v = dstbuf[j, pl.ds(g * 16, 16)]
                    idxbuf[pl.ds(g * 16, 16)] = v + boff
                pltpu.sync_copy(xs_hbm.at[idxbuf], rows)
                pltpu.sync_copy(rows, accum.at[srcbuf.at[j]], add=True)
            return carry

        lax.fori_loop(0, CPT // 8, body, 0, unroll=False)
        plsc.subcore_barrier()
        pltpu.sync_copy(accum.at[pl.ds(s * RPT, RPT)],
                        neigh_hbm.at[b * NS + s])
        plsc.subcore_barrier()


def _prep_body(x_ref, degp_ref, xs_ref):
    deg = jnp.sum(degp_ref[...], axis=1)
    inv = 1.0 / jnp.maximum(deg, 1.0)
    xs_ref[...] = x_ref[...] * inv[None, :, None]


def _prep(x, degp_n32):
    nb = 1000
    return pl.pallas_call(
        _prep_body,
        out_shape=jax.ShapeDtypeStruct((B, N, D), jnp.float32),
        grid=(N // nb,),
        in_specs=[
            pl.BlockSpec((B, nb, D), lambda i: (0, i, 0)),
            pl.BlockSpec((nb, NC * 128), lambda i: (i, 0)),
        ],
        out_specs=pl.BlockSpec((B, nb, D), lambda i: (0, i, 0)),
    )(x, degp_n32)


def _finish_body(neigh_ref, x_ref, wt_ref, b_ref, g_ref, be_ref, out_ref):
    h = jnp.dot(neigh_ref[0], wt_ref[...],
                preferred_element_type=jnp.float32) + b_ref[...]
    y = x_ref[0] + h
    mu = jnp.mean(y, axis=-1, keepdims=True)
    var = jnp.mean((y - mu) ** 2, axis=-1, keepdims=True)
    out_ref[0] = (y - mu) * lax.rsqrt(var + 1e-5) * g_ref[...] + be_ref[...]


def _finish(neigh, x, Wt, b2, g2, be2):
    nb = 1000
    return pl.pallas_call(
        _finish_body,
        out_shape=jax.ShapeDtypeStruct((B, N, D), jnp.float32),
        grid=(B, N // nb),
        in_specs=[
            pl.BlockSpec((1, nb, D), lambda i, j: (i, j, 0)),
            pl.BlockSpec((1, nb, D), lambda i, j: (i, j, 0)),
            pl.BlockSpec((D, D), lambda i, j: (0, 0)),
            pl.BlockSpec((1, D), lambda i, j: (0, 0)),
            pl.BlockSpec((1, D), lambda i, j: (0, 0)),
            pl.BlockSpec((1, D), lambda i, j: (0, 0)),
        ],
        out_specs=pl.BlockSpec((1, nb, D), lambda i, j: (i, j, 0)),
    )(neigh, x, Wt, b2, g2, be2)


def kernel(x, edge_index, W, b, gamma, beta):
    npad = EPAD - E
    src2d = jnp.concatenate(
        [edge_index[0], jnp.full((npad,), N, jnp.int32)]).reshape(ROWS, G)
    dst2d = jnp.concatenate(
        [edge_index[1], jnp.zeros((npad,), jnp.int32)]).reshape(ROWS, G)
    ones = jnp.full((G, 128), 1.0 / 128.0, jnp.float32)
    zrow = jnp.zeros((RPT, 128), jnp.float32)
    zeros = jnp.zeros((RPT, D), jnp.float32)

    degp = _deg_kernel(src2d, ones, zrow)
    degp_n32 = (degp.reshape(NC, NS, RPT, 128)
                .transpose(1, 2, 0, 3).reshape(NP, NC * 128)[:N])
    xs = _prep(x, degp_n32)

    neigh_pw = _agg_kernel(xs.reshape(B * N, D), src2d, dst2d, zeros)
    neigh = neigh_pw.reshape(B, NP, D)[:, :N]

    return _finish(neigh, x, W.T,
                   b.reshape(1, D), gamma.reshape(1, D), beta.reshape(1, D))


# speedup vs baseline: 2.7679x; 1.1530x over previous
import functools

import jax
import jax.numpy as jnp
from jax import lax
from jax.experimental import pallas as pl
from jax.experimental.pallas import tpu as pltpu
from jax.experimental.pallas import tpu_sc as plsc

B = 8
N = 10000
E = 320000
D = 128

NC = 2
NS = 16
G = 128
EPAD = 327680
ROWS = EPAD // G
CPT = ROWS // NS
DEG_CPT = ROWS // NC // NS
NP = 10112
RPT = NP // NS
BPC = B // NC

_mesh = plsc.VectorSubcoreMesh(core_axis_name="c", subcore_axis_name="s",
                               num_cores=NC, num_subcores=NS)


@functools.partial(
    pl.kernel,
    out_type=jax.ShapeDtypeStruct((NC * NS, RPT, 128), jnp.float32),
    mesh=_mesh,
    scratch_types=[
        pltpu.VMEM((8, G), jnp.int32),
        pltpu.VMEM((G, 128), jnp.float32),
        pltpu.VMEM_SHARED((NP, 128), jnp.float32),
    ],
)
def _deg_kernel(src2d_hbm, ones_hbm, zrow_hbm, degp_hbm, srcbuf, onesbuf,
                hist):
    c = lax.axis_index("c")
    s = lax.axis_index("s")
    w = c * NS + s
    pltpu.sync_copy(zrow_hbm, hist.at[pl.ds(s * RPT, RPT)])
    base = c * (NS * DEG_CPT) + s * DEG_CPT
    pltpu.sync_copy(ones_hbm, onesbuf)
    plsc.subcore_barrier()

    def body(grp, carry):
        pltpu.sync_copy(src2d_hbm.at[pl.ds(base + grp * 8, 8)], srcbuf)
        for j in range(8):
            pltpu.sync_copy(onesbuf, hist.at[srcbuf.at[j]], add=True)
        return carry

    lax.fori_loop(0, DEG_CPT // 8, body, 0, unroll=False)
    plsc.subcore_barrier()
    pltpu.sync_copy(hist.at[pl.ds(s * RPT, RPT)], degp_hbm.at[w])


@functools.partial(
    pl.kernel,
    out_type=jax.ShapeDtypeStruct((B * NS, RPT, D), jnp.float32),
    mesh=_mesh,
    scratch_types=[
        pltpu.VMEM((8, G), jnp.int32),
        pltpu.VMEM((8, G), jnp.int32),
        pltpu.VMEM((8, G), jnp.int32),
        pltpu.VMEM((G, D), jnp.float32),
        pltpu.VMEM((G, D), jnp.float32),
        pltpu.SemaphoreType.DMA,
        pltpu.SemaphoreType.DMA,
        pltpu.VMEM_SHARED((NP, D), jnp.float32),
    ],
)
def _agg_kernel(xs_hbm, src2d_hbm, dst2d_hbm, zeros_hbm, neigh_hbm,
                srcbuf, dstbuf, idx2d, rows0, rows1, sem0, sem1, accum):
    c = lax.axis_index("c")
    s = lax.axis_index("s")
    rows = (rows0, rows1)
    sems = (sem0, sem1)

    for k in range(BPC):
        b = c * BPC + k
        boff = b * N
        pltpu.sync_copy(zeros_hbm, accum.at[pl.ds(s * RPT, RPT)])
        plsc.subcore_barrier()

        def body(grp, carry):
            row0 = s * CPT + grp * 8
            pltpu.sync_copy(src2d_hbm.at[pl.ds(row0, 8)], srcbuf)
            pltpu.sync_copy(dst2d_hbm.at[pl.ds(row0, 8)], dstbuf)

            def compute_idx(jj):
                for g in range(G // 16):
                    v = dstbuf[jj, pl.ds(g * 16, 16)]
                    idx2d[jj, pl.ds(g * 16, 16)] = v + boff

            compute_idx(0)
            handles = [None] * 8
            handles[0] = pltpu.async_copy(
                xs_hbm.at[idx2d.at[0]], rows[0], sems[0])
            for j in range(8):
                if j < 7:
                    compute_idx(j + 1)
                    handles[j + 1] = pltpu.async_copy(
                        xs_hbm.at[idx2d.at[j + 1]], rows[(j + 1) % 2],
                        sems[(j + 1) % 2])
                handles[j].wait()
                pltpu.sync_copy(rows[j % 2], accum.at[srcbuf.at[j]],
                                add=True)
            return carry

        lax.fori_loop(0, CPT // 8, body, 0, unroll=False)
        plsc.subcore_barrier()
        pltpu.sync_copy(accum.at[pl.ds(s * RPT, RPT)],
                        neigh_hbm.at[b * NS + s])
        plsc.subcore_barrier()


def _prep_body(x_ref, degp_ref, xs_ref):
    deg = jnp.sum(degp_ref[...], axis=1)
    inv = 1.0 / jnp.maximum(deg, 1.0)
    xs_ref[...] = x_ref[...] * inv[None, :, None]


def _prep(x, degp_n32):
    nb = 1000
    return pl.pallas_call(
        _prep_body,
        out_shape=jax.ShapeDtypeStruct((B, N, D), jnp.float32),
        grid=(N // nb,),
        in_specs=[
            pl.BlockSpec((B, nb, D), lambda i: (0, i, 0)),
            pl.BlockSpec((nb, NC * 128), lambda i: (i, 0)),
        ],
        out_specs=pl.BlockSpec((B, nb, D), lambda i: (0, i, 0)),
    )(x, degp_n32)


def _finish_body(neigh_ref, x_ref, wt_ref, b_ref, g_ref, be_ref, out_ref):
    h = jnp.dot(neigh_ref[0], wt_ref[...],
                preferred_element_type=jnp.float32) + b_ref[...]
    y = x_ref[0] + h
    mu = jnp.mean(y, axis=-1, keepdims=True)
    var = jnp.mean((y - mu) ** 2, axis=-1, keepdims=True)
    out_ref[0] = (y - mu) * lax.rsqrt(var + 1e-5) * g_ref[...] + be_ref[...]


def _finish(neigh, x, Wt, b2, g2, be2):
    nb = 1000
    return pl.pallas_call(
        _finish_body,
        out_shape=jax.ShapeDtypeStruct((B, N, D), jnp.float32),
        grid=(B, N // nb),
        in_specs=[
            pl.BlockSpec((1, nb, D), lambda i, j: (i, j, 0)),
            pl.BlockSpec((1, nb, D), lambda i, j: (i, j, 0)),
            pl.BlockSpec((D, D), lambda i, j: (0, 0)),
            pl.BlockSpec((1, D), lambda i, j: (0, 0)),
            pl.BlockSpec((1, D), lambda i, j: (0, 0)),
            pl.BlockSpec((1, D), lambda i, j: (0, 0)),
        ],
        out_specs=pl.BlockSpec((1, nb, D), lambda i, j: (i, j, 0)),
    )(neigh, x, Wt, b2, g2, be2)


def kernel(x, edge_index, W, b, gamma, beta):
    npad = EPAD - E
    src2d = jnp.concatenate(
        [edge_index[0], jnp.full((npad,), N, jnp.int32)]).reshape(ROWS, G)
    dst2d = jnp.concatenate(
        [edge_index[1], jnp.zeros((npad,), jnp.int32)]).reshape(ROWS, G)
    ones = jnp.full((G, 128), 1.0 / 128.0, jnp.float32)
    zrow = jnp.zeros((RPT, 128), jnp.float32)
    zeros = jnp.zeros((RPT, D), jnp.float32)

    degp = _deg_kernel(src2d, ones, zrow)
    degp_n32 = (degp.reshape(NC, NS, RPT, 128)
                .transpose(1, 2, 0, 3).reshape(NP, NC * 128)[:N])
    xs = _prep(x, degp_n32)

    neigh_pw = _agg_kernel(xs.reshape(B * N, D), src2d, dst2d, zeros)
    neigh = neigh_pw.reshape(B, NP, D)[:, :N]

    return _finish(neigh, x, W.T,
                   b.reshape(1, D), gamma.reshape(1, D), beta.reshape(1, D))


# speedup vs baseline: 2.9270x; 1.0575x over previous
import functools

import jax
import jax.numpy as jnp
from jax import lax
from jax.experimental import pallas as pl
from jax.experimental.pallas import tpu as pltpu
from jax.experimental.pallas import tpu_sc as plsc

B = 8
N = 10000
E = 320000
D = 128

NC = 2
NS = 16
G = 128
EPAD = 327680
ROWS = EPAD // G
CPT = ROWS // NS
DEG_CPT = ROWS // NC // NS
NP = 10112
RPT = NP // NS
BPC = B // NC

_mesh = plsc.VectorSubcoreMesh(core_axis_name="c", subcore_axis_name="s",
                               num_cores=NC, num_subcores=NS)


@functools.partial(
    pl.kernel,
    out_type=jax.ShapeDtypeStruct((NC * NS, RPT, 128), jnp.float32),
    mesh=_mesh,
    scratch_types=[
        pltpu.VMEM((8, G), jnp.int32),
        pltpu.VMEM((G, 128), jnp.float32),
        pltpu.VMEM_SHARED((NP, 128), jnp.float32),
    ],
)
def _deg_kernel(src2d_hbm, ones_hbm, zrow_hbm, degp_hbm, srcbuf, onesbuf,
                hist):
    c = lax.axis_index("c")
    s = lax.axis_index("s")
    w = c * NS + s
    pltpu.sync_copy(zrow_hbm, hist.at[pl.ds(s * RPT, RPT)])
    base = c * (NS * DEG_CPT) + s * DEG_CPT
    pltpu.sync_copy(ones_hbm, onesbuf)
    plsc.subcore_barrier()

    def body(grp, carry):
        pltpu.sync_copy(src2d_hbm.at[pl.ds(base + grp * 8, 8)], srcbuf)
        for j in range(8):
            pltpu.sync_copy(onesbuf, hist.at[srcbuf.at[j]], add=True)
        return carry

    lax.fori_loop(0, DEG_CPT // 8, body, 0, unroll=False)
    plsc.subcore_barrier()
    pltpu.sync_copy(hist.at[pl.ds(s * RPT, RPT)], degp_hbm.at[w])


@functools.partial(
    pl.kernel,
    out_type=jax.ShapeDtypeStruct((B * NS, RPT, D), jnp.float32),
    mesh=_mesh,
    scratch_types=[
        pltpu.VMEM((40, G), jnp.int32),
        pltpu.VMEM((40, G), jnp.int32),
        pltpu.VMEM((2, G), jnp.int32),
        pltpu.VMEM((G, D), jnp.float32),
        pltpu.VMEM((G, D), jnp.float32),
        pltpu.SemaphoreType.DMA,
        pltpu.SemaphoreType.DMA,
        pltpu.VMEM_SHARED((NP, D), jnp.float32),
    ],
)
def _agg_kernel(xs_hbm, src2d_hbm, dst2d_hbm, zeros_hbm, neigh_hbm,
                srcbuf, dstbuf, idx2d, rows0, rows1, sem0, sem1, accum):
    c = lax.axis_index("c")
    s = lax.axis_index("s")
    rows = (rows0, rows1)
    sems = (sem0, sem1)

    def batch(k, carry):
        b = c * BPC + k
        boff = b * N
        pltpu.sync_copy(zeros_hbm, accum.at[pl.ds(s * RPT, RPT)])
        plsc.subcore_barrier()

        for grp in range(4):
            row0 = s * CPT + grp * 40
            pltpu.sync_copy(src2d_hbm.at[pl.ds(row0, 40)], srcbuf)
            pltpu.sync_copy(dst2d_hbm.at[pl.ds(row0, 40)], dstbuf)

            def compute_idx(jj):
                for g in range(G // 16):
                    v = dstbuf[jj, pl.ds(g * 16, 16)]
                    idx2d[jj % 2, pl.ds(g * 16, 16)] = v + boff

            compute_idx(0)
            handles = [None] * 40
            handles[0] = pltpu.async_copy(
                xs_hbm.at[idx2d.at[0]], rows[0], sems[0])
            for j in range(40):
                if j < 39:
                    compute_idx(j + 1)
                    handles[j + 1] = pltpu.async_copy(
                        xs_hbm.at[idx2d.at[(j + 1) % 2]], rows[(j + 1) % 2],
                        sems[(j + 1) % 2])
                handles[j].wait()
                pltpu.sync_copy(rows[j % 2], accum.at[srcbuf.at[j]],
                                add=True)

        plsc.subcore_barrier()
        pltpu.sync_copy(accum.at[pl.ds(s * RPT, RPT)],
                        neigh_hbm.at[b * NS + s])
        plsc.subcore_barrier()
        return carry

    lax.fori_loop(0, BPC, batch, 0, unroll=False)


def _prep_body(x_ref, degp_ref, xs_ref):
    deg = jnp.sum(degp_ref[...], axis=1)
    inv = 1.0 / jnp.maximum(deg, 1.0)
    xs_ref[...] = x_ref[...] * inv[None, :, None]


def _prep(x, degp_n32):
    nb = 1000
    return pl.pallas_call(
        _prep_body,
        out_shape=jax.ShapeDtypeStruct((B, N, D), jnp.float32),
        grid=(N // nb,),
        in_specs=[
            pl.BlockSpec((B, nb, D), lambda i: (0, i, 0)),
            pl.BlockSpec((nb, NC * 128), lambda i: (i, 0)),
        ],
        out_specs=pl.BlockSpec((B, nb, D), lambda i: (0, i, 0)),
    )(x, degp_n32)


def _finish_body(neigh_ref, x_ref, wt_ref, b_ref, g_ref, be_ref, out_ref):
    h = jnp.dot(neigh_ref[0], wt_ref[...],
                preferred_element_type=jnp.float32) + b_ref[...]
    y = x_ref[0] + h
    mu = jnp.mean(y, axis=-1, keepdims=True)
    var = jnp.mean((y - mu) ** 2, axis=-1, keepdims=True)
    out_ref[0] = (y - mu) * lax.rsqrt(var + 1e-5) * g_ref[...] + be_ref[...]


def _finish(neigh, x, Wt, b2, g2, be2):
    nb = 1000
    return pl.pallas_call(
        _finish_body,
        out_shape=jax.ShapeDtypeStruct((B, N, D), jnp.float32),
        grid=(B, N // nb),
        in_specs=[
            pl.BlockSpec((1, nb, D), lambda i, j: (i, j, 0)),
            pl.BlockSpec((1, nb, D), lambda i, j: (i, j, 0)),
            pl.BlockSpec((D, D), lambda i, j: (0, 0)),
            pl.BlockSpec((1, D), lambda i, j: (0, 0)),
            pl.BlockSpec((1, D), lambda i, j: (0, 0)),
            pl.BlockSpec((1, D), lambda i, j: (0, 0)),
        ],
        out_specs=pl.BlockSpec((1, nb, D), lambda i, j: (i, j, 0)),
    )(neigh, x, Wt, b2, g2, be2)


def kernel(x, edge_index, W, b, gamma, beta):
    npad = EPAD - E
    src2d = jnp.concatenate(
        [edge_index[0], jnp.full((npad,), N, jnp.int32)]).reshape(ROWS, G)
    dst2d = jnp.concatenate(
        [edge_index[1], jnp.zeros((npad,), jnp.int32)]).reshape(ROWS, G)
    ones = jnp.full((G, 128), 1.0 / 128.0, jnp.float32)
    zrow = jnp.zeros((RPT, 128), jnp.float32)
    zeros = jnp.zeros((RPT, D), jnp.float32)

    degp = _deg_kernel(src2d, ones, zrow)
    degp_n32 = (degp.reshape(NC, NS, RPT, 128)
                .transpose(1, 2, 0, 3).reshape(NP, NC * 128)[:N])
    xs = _prep(x, degp_n32)

    neigh_pw = _agg_kernel(xs.reshape(B * N, D), src2d, dst2d, zeros)
    neigh = neigh_pw.reshape(B, NP, D)[:, :N]

    return _finish(neigh, x, W.T,
                   b.reshape(1, D), gamma.reshape(1, D), beta.reshape(1, D))


# speedup vs baseline: 2.9350x; 1.0027x over previous
import functools

import jax
import jax.numpy as jnp
from jax import lax
from jax.experimental import pallas as pl
from jax.experimental.pallas import tpu as pltpu
from jax.experimental.pallas import tpu_sc as plsc

B = 8
N = 10000
E = 320000
D = 128

NC = 2
NS = 16
G = 128
EPAD = 327680
ROWS = EPAD // G
CPT = ROWS // NS
DEG_CPT = ROWS // NC // NS
NP = 10112
RPT = NP // NS
BPC = B // NC

_mesh = plsc.VectorSubcoreMesh(core_axis_name="c", subcore_axis_name="s",
                               num_cores=NC, num_subcores=NS)


@functools.partial(
    pl.kernel,
    out_type=jax.ShapeDtypeStruct((NC * NS, RPT, 128), jnp.float32),
    mesh=_mesh,
    scratch_types=[
        pltpu.VMEM((8, G), jnp.int32),
        pltpu.VMEM((G, 128), jnp.float32),
        pltpu.VMEM_SHARED((NP, 128), jnp.float32),
    ],
)
def _deg_kernel(src2d_hbm, ones_hbm, zrow_hbm, degp_hbm, srcbuf, onesbuf,
                hist):
    c = lax.axis_index("c")
    s = lax.axis_index("s")
    w = c * NS + s
    pltpu.sync_copy(zrow_hbm, hist.at[pl.ds(s * RPT, RPT)])
    base = c * (NS * DEG_CPT) + s * DEG_CPT
    pltpu.sync_copy(ones_hbm, onesbuf)
    plsc.subcore_barrier()

    def body(grp, carry):
        pltpu.sync_copy(src2d_hbm.at[pl.ds(base + grp * 8, 8)], srcbuf)
        for j in range(8):
            pltpu.sync_copy(onesbuf, hist.at[srcbuf.at[j]], add=True)
        return carry

    lax.fori_loop(0, DEG_CPT // 8, body, 0, unroll=False)
    plsc.subcore_barrier()
    pltpu.sync_copy(hist.at[pl.ds(s * RPT, RPT)], degp_hbm.at[w])


@functools.partial(
    pl.kernel,
    out_type=jax.ShapeDtypeStruct((B * NS, RPT, D), jnp.float32),
    mesh=_mesh,
    scratch_types=[
        pltpu.VMEM((2, 32, G), jnp.int32),
        pltpu.VMEM((2, 32, G), jnp.int32),
        pltpu.VMEM((2, G), jnp.int32),
        pltpu.VMEM((G, D), jnp.float32),
        pltpu.VMEM((G, D), jnp.float32),
        pltpu.SemaphoreType.DMA,
        pltpu.SemaphoreType.DMA,
        pltpu.SemaphoreType.DMA,
        pltpu.SemaphoreType.DMA,
        pltpu.VMEM_SHARED((NP, D), jnp.float32),
    ],
)
def _agg_kernel(xs_hbm, src2d_hbm, dst2d_hbm, zeros_hbm, neigh_hbm,
                srcbuf, dstbuf, idx2d, rows0, rows1, sem0, sem1, isem0,
                isem1, accum):
    c = lax.axis_index("c")
    s = lax.axis_index("s")
    rows = (rows0, rows1)
    sems = (sem0, sem1)
    isems = (isem0, isem1)

    def stage(grp, pb):
        row0 = s * CPT + grp * 32
        hs = pltpu.async_copy(src2d_hbm.at[pl.ds(row0, 32)],
                              srcbuf.at[pb], isems[pb])
        hd = pltpu.async_copy(dst2d_hbm.at[pl.ds(row0, 32)],
                              dstbuf.at[pb], isems[pb])
        return (hs, hd)

    def batch(k, carry):
        b = c * BPC + k
        boff = b * N
        pltpu.sync_copy(zeros_hbm, accum.at[pl.ds(s * RPT, RPT)])
        h0 = stage(0, 0)
        plsc.subcore_barrier()

        pend = h0
        for grp in range(5):
            pb = grp % 2
            pend[0].wait()
            pend[1].wait()
            if grp < 4:
                pend = stage(grp + 1, 1 - pb)

            def compute_idx(jj):
                for g in range(G // 16):
                    v = dstbuf[pb, jj, pl.ds(g * 16, 16)]
                    idx2d[jj % 2, pl.ds(g * 16, 16)] = v + boff

            compute_idx(0)
            handles = [None] * 32
            handles[0] = pltpu.async_copy(
                xs_hbm.at[idx2d.at[0]], rows[0], sems[0])
            for j in range(32):
                if j < 31:
                    compute_idx(j + 1)
                    handles[j + 1] = pltpu.async_copy(
                        xs_hbm.at[idx2d.at[(j + 1) % 2]], rows[(j + 1) % 2],
                        sems[(j + 1) % 2])
                handles[j].wait()
                pltpu.sync_copy(rows[j % 2], accum.at[srcbuf.at[pb, j]],
                                add=True)

        plsc.subcore_barrier()
        pltpu.sync_copy(accum.at[pl.ds(s * RPT, RPT)],
                        neigh_hbm.at[b * NS + s])
        plsc.subcore_barrier()
        return carry

    lax.fori_loop(0, BPC, batch, 0, unroll=False)


def _prep_body(x_ref, degp_ref, xs_ref):
    deg = jnp.sum(degp_ref[...], axis=1)
    inv = 1.0 / jnp.maximum(deg, 1.0)
    xs_ref[...] = x_ref[...] * inv[None, :, None]


def _prep(x, degp_n32):
    nb = 1000
    return pl.pallas_call(
        _prep_body,
        out_shape=jax.ShapeDtypeStruct((B, N, D), jnp.float32),
        grid=(N // nb,),
        in_specs=[
            pl.BlockSpec((B, nb, D), lambda i: (0, i, 0)),
            pl.BlockSpec((nb, NC * 128), lambda i: (i, 0)),
        ],
        out_specs=pl.BlockSpec((B, nb, D), lambda i: (0, i, 0)),
    )(x, degp_n32)


def _finish_body(neigh_ref, x_ref, wt_ref, b_ref, g_ref, be_ref, out_ref):
    h = jnp.dot(neigh_ref[0], wt_ref[...],
                preferred_element_type=jnp.float32) + b_ref[...]
    y = x_ref[0] + h
    mu = jnp.mean(y, axis=-1, keepdims=True)
    var = jnp.mean((y - mu) ** 2, axis=-1, keepdims=True)
    out_ref[0] = (y - mu) * lax.rsqrt(var + 1e-5) * g_ref[...] + be_ref[...]


def _finish(neigh, x, Wt, b2, g2, be2):
    nb = 1000
    return pl.pallas_call(
        _finish_body,
        out_shape=jax.ShapeDtypeStruct((B, N, D), jnp.float32),
        grid=(B, N // nb),
        in_specs=[
            pl.BlockSpec((1, nb, D), lambda i, j: (i, j, 0)),
            pl.BlockSpec((1, nb, D), lambda i, j: (i, j, 0)),
            pl.BlockSpec((D, D), lambda i, j: (0, 0)),
            pl.BlockSpec((1, D), lambda i, j: (0, 0)),
            pl.BlockSpec((1, D), lambda i, j: (0, 0)),
            pl.BlockSpec((1, D), lambda i, j: (0, 0)),
        ],
        out_specs=pl.BlockSpec((1, nb, D), lambda i, j: (i, j, 0)),
    )(neigh, x, Wt, b2, g2, be2)


def kernel(x, edge_index, W, b, gamma, beta):
    npad = EPAD - E
    src2d = jnp.concatenate(
        [edge_index[0], jnp.full((npad,), N, jnp.int32)]).reshape(ROWS, G)
    dst2d = jnp.concatenate(
        [edge_index[1], jnp.zeros((npad,), jnp.int32)]).reshape(ROWS, G)
    ones = jnp.full((G, 128), 1.0 / 128.0, jnp.float32)
    zrow = jnp.zeros((RPT, 128), jnp.float32)
    zeros = jnp.zeros((RPT, D), jnp.float32)

    degp = _deg_kernel(src2d, ones, zrow)
    degp_n32 = (degp.reshape(NC, NS, RPT, 128)
                .transpose(1, 2, 0, 3).reshape(NP, NC * 128)[:N])
    xs = _prep(x, degp_n32)

    neigh_pw = _agg_kernel(xs.reshape(B * N, D), src2d, dst2d, zeros)
    neigh = neigh_pw.reshape(B, NP, D)[:, :N]

    return _finish(neigh, x, W.T,
                   b.reshape(1, D), gamma.reshape(1, D), beta.reshape(1, D))
